# Initial kernel scaffold; baseline (speedup 1.0000x reference)
import jax
import jax.numpy as jnp
from jax.experimental import pallas as pl


def kernel(x, edge_index, batch, Wl_first, bl_first, Wr_first, Wl_common, bl_common, Wr_common, Wl_actor, bl_actor, Wr_actor, Wl_critic, bl_critic, Wr_critic, Wa, ba, Wc, bc):
    raise NotImplementedError("write your pallas kernel here")


# speedup vs baseline: 7.0085x; 7.0085x over previous
import functools

import jax
import jax.numpy as jnp
from jax import lax
from jax.experimental import pallas as pl
from jax.experimental.pallas import tpu as pltpu
from jax.experimental.pallas import tpu_sc as plsc

N = 512
U = 128
E = 8192

NC = 2
NS = 16
NW = NC * NS
EPW = E // NW
CHUNK = 128
NCHUNK = EPW // CHUNK
RPS = N // NS
DW = 128


def _sc_agg_body(with_deg, *args):
  if with_deg:
    (x_hbm, ei_hbm, z_hbm, ones_hbm, s_out, deg_out,
     sidx, didx, rows, ones_v, acc, dacc, sem) = args
  else:
    (x_hbm, ei_hbm, z_hbm, ones_hbm, s_out,
     sidx, didx, rows, acc, sem) = args
    deg_out = ones_v = dacc = None
  c = lax.axis_index("c")
  s = lax.axis_index("s")
  wid = s * NC + c
  row0 = s * RPS

  pltpu.sync_copy(z_hbm.at[pl.ds(row0, RPS)], acc.at[pl.ds(row0, RPS)])
  if with_deg:
    pltpu.sync_copy(z_hbm.at[pl.ds(row0, RPS)], dacc.at[pl.ds(row0, RPS)])
    pltpu.sync_copy(ones_hbm, ones_v)

  base = wid * EPW
  for j in range(NCHUNK):
    pltpu.sync_copy(ei_hbm.at[0, pl.ds(base + j * CHUNK, CHUNK)], sidx.at[j])
    pltpu.sync_copy(ei_hbm.at[1, pl.ds(base + j * CHUNK, CHUNK)], didx.at[j])

  plsc.subcore_barrier()

  for j in range(NCHUNK):
    pltpu.async_copy(x_hbm.at[sidx.at[j]], rows.at[j], sem).wait()
    pltpu.sync_copy(rows.at[j], acc.at[didx.at[j]], add=True)
    if with_deg:
      pltpu.sync_copy(ones_v, dacc.at[didx.at[j]], add=True)

  plsc.subcore_barrier()

  pltpu.sync_copy(acc.at[pl.ds(row0, RPS)], s_out.at[c, pl.ds(row0, RPS)])
  if with_deg:
    pltpu.sync_copy(dacc.at[pl.ds(row0, RPS)], deg_out.at[c, pl.ds(row0, RPS)])


@functools.lru_cache(maxsize=None)
def _make_sc_agg(with_deg):
  out_type = [jax.ShapeDtypeStruct((NC, N, U), jnp.float32)]
  if with_deg:
    out_type.append(jax.ShapeDtypeStruct((NC, N, DW), jnp.float32))
  scratch = [
      pltpu.VMEM((NCHUNK, CHUNK), jnp.int32),
      pltpu.VMEM((NCHUNK, CHUNK), jnp.int32),
      pltpu.VMEM((NCHUNK, CHUNK, U), jnp.float32),
  ]
  if with_deg:
    scratch.append(pltpu.VMEM((CHUNK, DW), jnp.float32))
  scratch.append(pltpu.VMEM_SHARED((N, U), jnp.float32))
  if with_deg:
    scratch.append(pltpu.VMEM_SHARED((N, DW), jnp.float32))
  scratch.append(pltpu.SemaphoreType.DMA)
  return pl.kernel(
      functools.partial(_sc_agg_body, with_deg),
      out_type=out_type,
      mesh=plsc.VectorSubcoreMesh(core_axis_name="c", subcore_axis_name="s",
                                  num_cores=NC, num_subcores=NS),
      scratch_types=scratch,
      name="sage_segment_sum" + ("_deg" if with_deg else ""),
  )


def _dotT(a, w):
  return lax.dot_general(a, w, (((1,), (1,)), ((), ())),
                         preferred_element_type=jnp.float32,
                         precision=lax.Precision.HIGHEST)


def _mean_from_parts(s_ref, degw_ref):
  deg = jnp.maximum(degw_ref[0, :, 0:1] + degw_ref[1, :, 0:1], 1.0)
  return (s_ref[0] + s_ref[1]) / deg


def _dense_body(s_ref, degw_ref, x_ref, wl_ref, bl_ref, wr_ref, o_ref):
  mean = _mean_from_parts(s_ref, degw_ref)
  o_ref[...] = jnp.tanh(_dotT(mean, wl_ref[...]) + bl_ref[...]
                        + _dotT(x_ref[...], wr_ref[...]))


def _final_body(s_ref, degw_ref, h_ref,
                wla_ref, bla_ref, wra_ref,
                wlc_ref, blc_ref, wrc_ref,
                wa_ref, wc_ref, bc_ref,
                ea_ref, ec_ref):
  mean = _mean_from_parts(s_ref, degw_ref)
  h = h_ref[...]
  xa = _dotT(mean, wla_ref[...]) + bla_ref[...] + _dotT(h, wra_ref[...])
  xc = _dotT(mean, wlc_ref[...]) + blc_ref[...] + _dotT(h, wrc_ref[...])
  wa = wa_ref[...]
  a = _dotT(xa, wa[:, :U])
  bt = _dotT(wa[:, U:], xa)
  cv = _dotT(xc, wc_ref[...])
  ma = jnp.max(a)
  mb = jnp.max(bt)
  lse = (ma + mb + jnp.log(jnp.sum(jnp.exp(a - ma)))
         + jnp.log(jnp.sum(jnp.exp(bt - mb))))
  ea_ref[...] = (a - lse) + bt
  ec_ref[...] = jnp.tanh(jnp.mean(cv) + bc_ref[0, 0]).reshape(1, 1)


def _dense(s_parts, degw, x, wl, bl, wr):
  return pl.pallas_call(
      _dense_body,
      out_shape=jax.ShapeDtypeStruct((N, U), jnp.float32),
  )(s_parts, degw, x, wl, bl.reshape(1, U), wr)


def kernel(x, edge_index, batch,
           Wl_first, bl_first, Wr_first,
           Wl_common, bl_common, Wr_common,
           Wl_actor, bl_actor, Wr_actor,
           Wl_critic, bl_critic, Wr_critic,
           Wa, ba, Wc, bc):
  ei = edge_index.astype(jnp.int32)
  zeros_h = jnp.zeros((N, U), jnp.float32)
  ones_h = jnp.ones((CHUNK, DW), jnp.float32)

  s_x, degw = _make_sc_agg(True)(x, ei, zeros_h, ones_h)
  h1 = _dense(s_x, degw, x, Wl_first, bl_first, Wr_first)
  (s_h1,) = _make_sc_agg(False)(h1, ei, zeros_h, ones_h)
  h = _dense(s_h1, degw, h1, Wl_common, bl_common, Wr_common)
  (s_h,) = _make_sc_agg(False)(h, ei, zeros_h, ones_h)

  ea, ec = pl.pallas_call(
      _final_body,
      out_shape=[
          jax.ShapeDtypeStruct((N, N), jnp.float32),
          jax.ShapeDtypeStruct((1, 1), jnp.float32),
      ],
  )(s_h, degw, h,
    Wl_actor, bl_actor.reshape(1, U), Wr_actor,
    Wl_critic, bl_critic.reshape(1, U), Wr_critic,
    Wa, Wc, bc.reshape(1, 1))

  return ea.reshape(N * N, 1), ec


# speedup vs baseline: 8.2043x; 1.1706x over previous
import functools

import jax
import jax.numpy as jnp
from jax import lax
from jax.experimental import pallas as pl
from jax.experimental.pallas import tpu as pltpu
from jax.experimental.pallas import tpu_sc as plsc

N = 512
U = 128
E = 8192

NC = 2
NS = 16
NW = NC * NS
EPW = E // NW
CHUNK = 128
NCHUNK = EPW // CHUNK
RPS = N // NS
DW = 128


def _sc_agg_body(with_deg, *args):
  if with_deg:
    (x_hbm, ei_hbm, z_hbm, ones_hbm, s_out, deg_out,
     eidx, rows, ones_v, acc, dacc, semz, semi, semg, sems, semr) = args
  else:
    (x_hbm, ei_hbm, z_hbm, ones_hbm, s_out,
     eidx, rows, acc, semz, semi, semg, sems, semr) = args
    deg_out = ones_v = dacc = None
  c = lax.axis_index("c")
  s = lax.axis_index("s")
  wid = s * NC + c
  row0 = s * RPS
  base = wid * EPW

  zc = [pltpu.async_copy(z_hbm.at[pl.ds(row0, RPS)],
                         acc.at[pl.ds(row0, RPS)], semz)]
  if with_deg:
    zc.append(pltpu.async_copy(z_hbm.at[pl.ds(row0, RPS)],
                               dacc.at[pl.ds(row0, RPS)], semz))
    zc.append(pltpu.async_copy(ones_hbm, ones_v, semz))
  ic = [pltpu.async_copy(
      ei_hbm.at[pl.ds(0, 2), pl.ds(base + j * CHUNK, CHUNK)], eidx.at[j], semi)
      for j in range(NCHUNK)]
  for cp in ic:
    cp.wait()
  gc = [pltpu.async_copy(x_hbm.at[eidx.at[j, 0]], rows.at[j], semg)
        for j in range(NCHUNK)]
  for cp in zc:
    cp.wait()
  plsc.subcore_barrier()
  for cp in gc:
    cp.wait()
  sc = []
  for j in range(NCHUNK):
    sc.append(pltpu.async_copy(rows.at[j], acc.at[eidx.at[j, 1]], sems,
                               add=True))
    if with_deg:
      sc.append(pltpu.async_copy(ones_v, dacc.at[eidx.at[j, 1]], sems,
                                 add=True))
  for cp in sc:
    cp.wait()
  plsc.subcore_barrier()
  rc = [pltpu.async_copy(acc.at[pl.ds(row0, RPS)],
                         s_out.at[c, pl.ds(row0, RPS)], semr)]
  if with_deg:
    rc.append(pltpu.async_copy(dacc.at[pl.ds(row0, RPS)],
                               deg_out.at[c, pl.ds(row0, RPS)], semr))
  for cp in rc:
    cp.wait()


@functools.lru_cache(maxsize=None)
def _make_sc_agg(with_deg):
  out_type = [jax.ShapeDtypeStruct((NC, N, U), jnp.float32)]
  if with_deg:
    out_type.append(jax.ShapeDtypeStruct((NC, N, DW), jnp.float32))
  scratch = [
      pltpu.VMEM((NCHUNK, 2, CHUNK), jnp.int32),
      pltpu.VMEM((NCHUNK, CHUNK, U), jnp.float32),
  ]
  if with_deg:
    scratch.append(pltpu.VMEM((CHUNK, DW), jnp.float32))
  scratch.append(pltpu.VMEM_SHARED((N, U), jnp.float32))
  if with_deg:
    scratch.append(pltpu.VMEM_SHARED((N, DW), jnp.float32))
  scratch.extend([pltpu.SemaphoreType.DMA] * 5)
  return pl.kernel(
      functools.partial(_sc_agg_body, with_deg),
      out_type=out_type,
      mesh=plsc.VectorSubcoreMesh(core_axis_name="c", subcore_axis_name="s",
                                  num_cores=NC, num_subcores=NS),
      scratch_types=scratch,
      name="sage_segment_sum" + ("_deg" if with_deg else ""),
  )


def _dotT(a, w):
  return lax.dot_general(a, w, (((1,), (1,)), ((), ())),
                         preferred_element_type=jnp.float32,
                         precision=lax.Precision.HIGHEST)


def _dense_first_body(s_ref, degw_ref, x_ref, wl_ref, bl_ref, wr_ref,
                      o_ref, deg_ref):
  deg = jnp.maximum(degw_ref[0, :, 0:1] + degw_ref[1, :, 0:1], 1.0)
  deg_ref[...] = deg
  mean = (s_ref[0] + s_ref[1]) / deg
  o_ref[...] = jnp.tanh(_dotT(mean, wl_ref[...]) + bl_ref[...]
                        + _dotT(x_ref[...], wr_ref[...]))


def _dense_body(s_ref, deg_ref, x_ref, wl_ref, bl_ref, wr_ref, o_ref):
  mean = (s_ref[0] + s_ref[1]) / deg_ref[...]
  o_ref[...] = jnp.tanh(_dotT(mean, wl_ref[...]) + bl_ref[...]
                        + _dotT(x_ref[...], wr_ref[...]))


def _final_body(s_ref, deg_ref, h_ref,
                wla_ref, bla_ref, wra_ref,
                wlc_ref, blc_ref, wrc_ref,
                wa_ref, wc_ref, bc_ref,
                ea_ref, ec_ref):
  mean = (s_ref[0] + s_ref[1]) / deg_ref[...]
  h = h_ref[...]
  xa = _dotT(mean, wla_ref[...]) + bla_ref[...] + _dotT(h, wra_ref[...])
  xc = _dotT(mean, wlc_ref[...]) + blc_ref[...] + _dotT(h, wrc_ref[...])
  wa = wa_ref[...]
  a = _dotT(xa, wa[:, :U])
  bt = _dotT(wa[:, U:], xa)
  cv = _dotT(xc, wc_ref[...])
  ma = jnp.max(a)
  mb = jnp.max(bt)
  lse = (ma + mb + jnp.log(jnp.sum(jnp.exp(a - ma)))
         + jnp.log(jnp.sum(jnp.exp(bt - mb))))
  ea_ref[...] = (a - lse) + bt
  ec_ref[...] = jnp.tanh(jnp.mean(cv) + bc_ref[0, 0]).reshape(1, 1)


def _dense(s_parts, deg, x, wl, bl, wr):
  return pl.pallas_call(
      _dense_body,
      out_shape=jax.ShapeDtypeStruct((N, U), jnp.float32),
  )(s_parts, deg, x, wl, bl.reshape(1, U), wr)


def kernel(x, edge_index, batch,
           Wl_first, bl_first, Wr_first,
           Wl_common, bl_common, Wr_common,
           Wl_actor, bl_actor, Wr_actor,
           Wl_critic, bl_critic, Wr_critic,
           Wa, ba, Wc, bc):
  ei = edge_index.astype(jnp.int32)
  zeros_h = jnp.zeros((N, U), jnp.float32)
  ones_h = jnp.ones((CHUNK, DW), jnp.float32)

  s_x, degw = _make_sc_agg(True)(x, ei, zeros_h, ones_h)
  h1, deg = pl.pallas_call(
      _dense_first_body,
      out_shape=[
          jax.ShapeDtypeStruct((N, U), jnp.float32),
          jax.ShapeDtypeStruct((N, 1), jnp.float32),
      ],
  )(s_x, degw, x, Wl_first, bl_first.reshape(1, U), Wr_first)
  (s_h1,) = _make_sc_agg(False)(h1, ei, zeros_h, ones_h)
  h = _dense(s_h1, deg, h1, Wl_common, bl_common, Wr_common)
  (s_h,) = _make_sc_agg(False)(h, ei, zeros_h, ones_h)

  ea, ec = pl.pallas_call(
      _final_body,
      out_shape=[
          jax.ShapeDtypeStruct((N, N), jnp.float32),
          jax.ShapeDtypeStruct((1, 1), jnp.float32),
      ],
  )(s_h, deg, h,
    Wl_actor, bl_actor.reshape(1, U), Wr_actor,
    Wl_critic, bl_critic.reshape(1, U), Wr_critic,
    Wa, Wc, bc.reshape(1, 1))

  return ea.reshape(N * N, 1), ec


# speedup vs baseline: 8.3384x; 1.0163x over previous
import functools

import jax
import jax.numpy as jnp
import numpy as np
from jax import lax
from jax.experimental import pallas as pl
from jax.experimental.pallas import tpu as pltpu
from jax.experimental.pallas import tpu_sc as plsc

N = 512
U = 128
E = 8192

NC = 2
NS = 16
NW = NC * NS
EPW = E // NW
CHUNK = 128
NCHUNK = EPW // CHUNK
RPS = N // NS
DW = 128


def _sc_agg_body(with_deg, *args):
  if with_deg:
    (x_hbm, ei_hbm, z_hbm, ones_hbm, s_out, deg_out,
     eidx, rows, ones_v, acc, dacc, semz, semi, semg, sems, semr) = args
  else:
    (x_hbm, ei_hbm, z_hbm, ones_hbm, s_out,
     eidx, rows, acc, semz, semi, semg, sems, semr) = args
    deg_out = ones_v = dacc = None
  c = lax.axis_index("c")
  s = lax.axis_index("s")
  wid = s * NC + c
  row0 = s * RPS
  base = wid * EPW

  zc = [pltpu.async_copy(z_hbm.at[pl.ds(row0, RPS)],
                         acc.at[pl.ds(row0, RPS)], semz)]
  if with_deg:
    zc.append(pltpu.async_copy(z_hbm.at[pl.ds(row0, RPS)],
                               dacc.at[pl.ds(row0, RPS)], semz))
    zc.append(pltpu.async_copy(ones_hbm, ones_v, semz))
  ic = [pltpu.async_copy(
      ei_hbm.at[pl.ds(0, 2), pl.ds(base + j * CHUNK, CHUNK)], eidx.at[j], semi)
      for j in range(NCHUNK)]
  for cp in ic:
    cp.wait()
  gc = [pltpu.async_copy(x_hbm.at[eidx.at[j, 0]], rows.at[j], semg)
        for j in range(NCHUNK)]
  for cp in zc:
    cp.wait()
  plsc.subcore_barrier()
  for cp in gc:
    cp.wait()
  sc = []
  for j in range(NCHUNK):
    sc.append(pltpu.async_copy(rows.at[j], acc.at[eidx.at[j, 1]], sems,
                               add=True))
    if with_deg:
      sc.append(pltpu.async_copy(ones_v, dacc.at[eidx.at[j, 1]], sems,
                                 add=True))
  for cp in sc:
    cp.wait()
  plsc.subcore_barrier()
  rc = [pltpu.async_copy(acc.at[pl.ds(row0, RPS)],
                         s_out.at[c, pl.ds(row0, RPS)], semr)]
  if with_deg:
    rc.append(pltpu.async_copy(dacc.at[pl.ds(row0, RPS)],
                               deg_out.at[c, pl.ds(row0, RPS)], semr))
  for cp in rc:
    cp.wait()


@functools.lru_cache(maxsize=None)
def _make_sc_agg(with_deg):
  out_type = [jax.ShapeDtypeStruct((NC, N, U), jnp.float32)]
  if with_deg:
    out_type.append(jax.ShapeDtypeStruct((NC, N, DW), jnp.float32))
  scratch = [
      pltpu.VMEM((NCHUNK, 2, CHUNK), jnp.int32),
      pltpu.VMEM((NCHUNK, CHUNK, U), jnp.float32),
  ]
  if with_deg:
    scratch.append(pltpu.VMEM((CHUNK, DW), jnp.float32))
  scratch.append(pltpu.VMEM_SHARED((N, U), jnp.float32))
  if with_deg:
    scratch.append(pltpu.VMEM_SHARED((N, DW), jnp.float32))
  scratch.extend([pltpu.SemaphoreType.DMA] * 5)
  return pl.kernel(
      functools.partial(_sc_agg_body, with_deg),
      out_type=out_type,
      mesh=plsc.VectorSubcoreMesh(core_axis_name="c", subcore_axis_name="s",
                                  num_cores=NC, num_subcores=NS),
      scratch_types=scratch,
      name="sage_segment_sum" + ("_deg" if with_deg else ""),
  )


def _dotT(a, w):
  return lax.dot_general(a, w, (((1,), (1,)), ((), ())),
                         preferred_element_type=jnp.float32,
                         precision=lax.Precision.HIGHEST)


def _dotN(a, w):
  return lax.dot_general(a, w, (((1,), (0,)), ((), ())),
                         preferred_element_type=jnp.float32,
                         precision=lax.Precision.HIGHEST)


def _dense_first_body(s_ref, degw_ref, x_ref, wl_ref, bl_ref, wr_ref,
                      o_ref, deg_ref):
  deg = jnp.maximum(degw_ref[0, :, 0:1] + degw_ref[1, :, 0:1], 1.0)
  deg_ref[...] = deg
  mean = (s_ref[0] + s_ref[1]) / deg
  o_ref[...] = jnp.tanh(_dotT(mean, wl_ref[...]) + bl_ref[...]
                        + _dotT(x_ref[...], wr_ref[...]))


def _dense_body(s_ref, deg_ref, x_ref, wl_ref, bl_ref, wr_ref, o_ref):
  mean = (s_ref[0] + s_ref[1]) / deg_ref[...]
  o_ref[...] = jnp.tanh(_dotT(mean, wl_ref[...]) + bl_ref[...]
                        + _dotT(x_ref[...], wr_ref[...]))


def _final_body(s_ref, deg_ref, h_ref,
                wla_ref, bla_ref, wra_ref,
                wlc_ref, blc_ref, wrc_ref,
                wa_ref, wc_ref, bc_ref, rrep_ref, ttile_ref,
                ea_ref, ec_ref):
  mean = (s_ref[0] + s_ref[1]) / deg_ref[...]
  h = h_ref[...]
  xa = _dotT(mean, wla_ref[...]) + bla_ref[...] + _dotT(h, wra_ref[...])
  xc = _dotT(mean, wlc_ref[...]) + blc_ref[...] + _dotT(h, wrc_ref[...])
  wa = wa_ref[...]
  a = _dotT(xa, wa[:, :U])
  b4 = jnp.concatenate(
      [_dotT(wa[:, U:], xa[k * 128:(k + 1) * 128, :]) for k in range(4)],
      axis=0)
  cv = _dotT(xc, wc_ref[...])
  ma = jnp.max(a)
  mb = jnp.max(b4)
  lse = (ma + mb + jnp.log(jnp.sum(jnp.exp(a - ma)))
         + jnp.log(jnp.sum(jnp.exp(b4 - mb))))
  a_shift = a - lse
  a_rep = jnp.concatenate(
      [_dotN(rrep_ref[...], a_shift[128 * m:128 * (m + 1), :])
       for m in range(4)], axis=0)
  b_tile = _dotN(ttile_ref[...], b4)
  ea_ref[...] = a_rep + b_tile
  ec_ref[...] = jnp.tanh(jnp.mean(cv) + bc_ref[0, 0]).reshape(1, 1)


@functools.lru_cache(maxsize=None)
def _rrep():
  r = np.zeros((N, 128), np.float32)
  r[np.arange(N), np.arange(N) // 4] = 1.0
  return r


@functools.lru_cache(maxsize=None)
def _ttile():
  t = np.zeros((4 * N, 4), np.float32)
  t[np.arange(4 * N), np.arange(4 * N) % 4] = 1.0
  return t


def _dense(s_parts, deg, x, wl, bl, wr):
  return pl.pallas_call(
      _dense_body,
      out_shape=jax.ShapeDtypeStruct((N, U), jnp.float32),
  )(s_parts, deg, x, wl, bl.reshape(1, U), wr)


def kernel(x, edge_index, batch,
           Wl_first, bl_first, Wr_first,
           Wl_common, bl_common, Wr_common,
           Wl_actor, bl_actor, Wr_actor,
           Wl_critic, bl_critic, Wr_critic,
           Wa, ba, Wc, bc):
  ei = edge_index.astype(jnp.int32)
  zeros_h = np.zeros((N, U), np.float32)
  ones_h = np.ones((CHUNK, DW), np.float32)

  s_x, degw = _make_sc_agg(True)(x, ei, zeros_h, ones_h)
  h1, deg = pl.pallas_call(
      _dense_first_body,
      out_shape=[
          jax.ShapeDtypeStruct((N, U), jnp.float32),
          jax.ShapeDtypeStruct((N, 1), jnp.float32),
      ],
  )(s_x, degw, x, Wl_first, bl_first.reshape(1, U), Wr_first)
  (s_h1,) = _make_sc_agg(False)(h1, ei, zeros_h, ones_h)
  h = _dense(s_h1, deg, h1, Wl_common, bl_common, Wr_common)
  (s_h,) = _make_sc_agg(False)(h, ei, zeros_h, ones_h)

  ea, ec = pl.pallas_call(
      _final_body,
      out_shape=[
          jax.ShapeDtypeStruct((4 * N, 128), jnp.float32),
          jax.ShapeDtypeStruct((1, 1), jnp.float32),
      ],
  )(s_h, deg, h,
    Wl_actor, bl_actor.reshape(1, U), Wr_actor,
    Wl_critic, bl_critic.reshape(1, U), Wr_critic,
    Wa, Wc, bc.reshape(1, 1), _rrep(), _ttile())

  return ea.reshape(N * N, 1), ec


# speedup vs baseline: 8.5973x; 1.0310x over previous
import functools

import jax
import jax.numpy as jnp
import numpy as np
from jax import lax
from jax.experimental import pallas as pl
from jax.experimental.pallas import tpu as pltpu
from jax.experimental.pallas import tpu_sc as plsc

N = 512
U = 128
E = 8192

NC = 2
NS = 16
NW = NC * NS
EPW = E // NW
CHUNK = 128
NCHUNK = EPW // CHUNK
RPS = N // NS
DW = 128


DN = 16


def _sc_agg_body(with_deg, *args):
  if with_deg:
    (x_hbm, ei_hbm, s_out, deg_out,
     eidx, rows, ones_v, zbuf, acc, dacc,
     semz, semi, semg, sems, semr) = args
  else:
    (x_hbm, ei_hbm, s_out,
     eidx, rows, zbuf, acc, semz, semi, semg, sems, semr) = args
    deg_out = ones_v = dacc = None
  c = lax.axis_index("c")
  s = lax.axis_index("s")
  wid = s * NC + c
  row0 = s * RPS
  base = wid * EPW

  ic = [pltpu.async_copy(
      ei_hbm.at[pl.ds(0, 2), pl.ds(base + j * CHUNK, CHUNK)], eidx.at[j], semi)
      for j in range(NCHUNK)]

  z16 = jnp.zeros((16,), jnp.float32)

  def zrow(i, _):
    for cc in range(U // 16):
      zbuf[i, pl.ds(cc * 16, 16)] = z16
    return 0

  lax.fori_loop(0, RPS, zrow, 0)
  if with_deg:
    o16 = jnp.ones((16,), jnp.float32)

    def orow(i, _):
      for cc in range(U // 16):
        ones_v[i, pl.ds(cc * 16, 16)] = o16
      return 0

    lax.fori_loop(0, CHUNK, orow, 0)

  zc = [pltpu.async_copy(zbuf, acc.at[pl.ds(row0, RPS)], semz)]
  if with_deg:
    zc.append(pltpu.async_copy(zbuf, dacc.at[pl.ds(row0, RPS)], semz))
  for cp in ic:
    cp.wait()
  gc = [pltpu.async_copy(x_hbm.at[eidx.at[j, 0]], rows.at[j], semg)
        for j in range(NCHUNK)]
  for cp in zc:
    cp.wait()
  plsc.subcore_barrier()
  for cp in gc:
    cp.wait()
  sc = []
  for j in range(NCHUNK):
    sc.append(pltpu.async_copy(rows.at[j], acc.at[eidx.at[j, 1]], sems,
                               add=True))
    if with_deg:
      sc.append(pltpu.async_copy(ones_v, dacc.at[eidx.at[j, 1]], sems,
                                 add=True))
  for cp in sc:
    cp.wait()
  plsc.subcore_barrier()
  rc = [pltpu.async_copy(acc.at[pl.ds(row0, RPS)],
                         s_out.at[c, pl.ds(row0, RPS)], semr)]
  if with_deg:
    rc.append(pltpu.async_copy(dacc.at[pl.ds(row0, RPS)],
                               deg_out.at[c, pl.ds(row0, RPS)], semr))
  for cp in rc:
    cp.wait()


@functools.lru_cache(maxsize=None)
def _make_sc_agg(with_deg):
  out_type = [jax.ShapeDtypeStruct((NC, N, U), jnp.float32)]
  if with_deg:
    out_type.append(jax.ShapeDtypeStruct((NC, N, DW), jnp.float32))
  scratch = [
      pltpu.VMEM((NCHUNK, 2, CHUNK), jnp.int32),
      pltpu.VMEM((NCHUNK, CHUNK, U), jnp.float32),
  ]
  if with_deg:
    scratch.append(pltpu.VMEM((CHUNK, DW), jnp.float32))
  scratch.append(pltpu.VMEM((RPS, U), jnp.float32))
  scratch.append(pltpu.VMEM_SHARED((N, U), jnp.float32))
  if with_deg:
    scratch.append(pltpu.VMEM_SHARED((N, DW), jnp.float32))
  scratch.extend([pltpu.SemaphoreType.DMA] * 5)
  return pl.kernel(
      functools.partial(_sc_agg_body, with_deg),
      out_type=out_type,
      mesh=plsc.VectorSubcoreMesh(core_axis_name="c", subcore_axis_name="s",
                                  num_cores=NC, num_subcores=NS),
      scratch_types=scratch,
      name="sage_segment_sum" + ("_deg" if with_deg else ""),
  )


def _dotT(a, w):
  return lax.dot_general(a, w, (((1,), (1,)), ((), ())),
                         preferred_element_type=jnp.float32,
                         precision=lax.Precision.HIGHEST)


def _dotN(a, w):
  return lax.dot_general(a, w, (((1,), (0,)), ((), ())),
                         preferred_element_type=jnp.float32,
                         precision=lax.Precision.HIGHEST)


def _dense_first_body(s_ref, degw_ref, x_ref, wl_ref, bl_ref, wr_ref,
                      o_ref, deg_ref):
  deg = jnp.maximum(degw_ref[0, :, 0:1] + degw_ref[1, :, 0:1], 1.0)
  deg_ref[...] = deg
  mean = (s_ref[0] + s_ref[1]) / deg
  o_ref[...] = jnp.tanh(_dotT(mean, wl_ref[...]) + bl_ref[...]
                        + _dotT(x_ref[...], wr_ref[...]))


def _dense_body(s_ref, deg_ref, x_ref, wl_ref, bl_ref, wr_ref, o_ref):
  mean = (s_ref[0] + s_ref[1]) / deg_ref[...]
  o_ref[...] = jnp.tanh(_dotT(mean, wl_ref[...]) + bl_ref[...]
                        + _dotT(x_ref[...], wr_ref[...]))


def _final_body(s_ref, deg_ref, h_ref,
                wla_ref, bla_ref, wra_ref,
                wlc_ref, blc_ref, wrc_ref,
                wa_ref, wc_ref, bc_ref, rrep_ref, ttile_ref,
                ea_ref, ec_ref):
  mean = (s_ref[0] + s_ref[1]) / deg_ref[...]
  h = h_ref[...]
  xa = _dotT(mean, wla_ref[...]) + bla_ref[...] + _dotT(h, wra_ref[...])
  xc = _dotT(mean, wlc_ref[...]) + blc_ref[...] + _dotT(h, wrc_ref[...])
  wa = wa_ref[...]
  a = _dotT(xa, wa[:, :U])
  b4 = jnp.concatenate(
      [_dotT(wa[:, U:], xa[k * 128:(k + 1) * 128, :]) for k in range(4)],
      axis=0)
  cv = _dotT(xc, wc_ref[...])
  ma = jnp.max(a)
  mb = jnp.max(b4)
  lse = (ma + mb + jnp.log(jnp.sum(jnp.exp(a - ma)))
         + jnp.log(jnp.sum(jnp.exp(b4 - mb))))
  a_shift = a - lse
  a_rep = jnp.concatenate(
      [_dotN(rrep_ref[...], a_shift[128 * m:128 * (m + 1), :])
       for m in range(4)], axis=0)
  b_tile = _dotN(ttile_ref[...], b4)
  ea_ref[...] = a_rep + b_tile
  ec_ref[...] = jnp.tanh(jnp.mean(cv) + bc_ref[0, 0]).reshape(1, 1)


@functools.lru_cache(maxsize=None)
def _rrep():
  r = np.zeros((N, 128), np.float32)
  r[np.arange(N), np.arange(N) // 4] = 1.0
  return r


@functools.lru_cache(maxsize=None)
def _ttile():
  t = np.zeros((4 * N, 4), np.float32)
  t[np.arange(4 * N), np.arange(4 * N) % 4] = 1.0
  return t


def _dense(s_parts, deg, x, wl, bl, wr):
  return pl.pallas_call(
      _dense_body,
      out_shape=jax.ShapeDtypeStruct((N, U), jnp.float32),
  )(s_parts, deg, x, wl, bl.reshape(1, U), wr)


def kernel(x, edge_index, batch,
           Wl_first, bl_first, Wr_first,
           Wl_common, bl_common, Wr_common,
           Wl_actor, bl_actor, Wr_actor,
           Wl_critic, bl_critic, Wr_critic,
           Wa, ba, Wc, bc):
  ei = edge_index.astype(jnp.int32)

  s_x, degw = _make_sc_agg(True)(x, ei)
  h1, deg = pl.pallas_call(
      _dense_first_body,
      out_shape=[
          jax.ShapeDtypeStruct((N, U), jnp.float32),
          jax.ShapeDtypeStruct((N, 1), jnp.float32),
      ],
  )(s_x, degw, x, Wl_first, bl_first.reshape(1, U), Wr_first)
  (s_h1,) = _make_sc_agg(False)(h1, ei)
  h = _dense(s_h1, deg, h1, Wl_common, bl_common, Wr_common)
  (s_h,) = _make_sc_agg(False)(h, ei)

  ea, ec = pl.pallas_call(
      _final_body,
      out_shape=[
          jax.ShapeDtypeStruct((4 * N, 128), jnp.float32),
          jax.ShapeDtypeStruct((1, 1), jnp.float32),
      ],
  )(s_h, deg, h,
    Wl_actor, bl_actor.reshape(1, U), Wr_actor,
    Wl_critic, bl_critic.reshape(1, U), Wr_critic,
    Wa, Wc, bc.reshape(1, 1), _rrep(), _ttile())

  return ea.reshape(N * N, 1), ec


# speedup vs baseline: 8.9458x; 1.0405x over previous
import functools

import jax
import jax.numpy as jnp
import numpy as np
from jax import lax
from jax.experimental import pallas as pl
from jax.experimental.pallas import tpu as pltpu
from jax.experimental.pallas import tpu_sc as plsc

N = 512
U = 128
E = 8192

NC = 2
NS = 16
NW = NC * NS
EPW = E // NW
CHUNK = 128
NCHUNK = EPW // CHUNK
RPS = N // NS
DW = 128


DN = 16


def _sc_agg_body(with_deg, *args):
  if with_deg:
    (x_hbm, ei_hbm, s_out, deg_out,
     eidx, rows, ones_v, zbuf, acc, dacc,
     semz, semi, semg0, semg1, sems, semr) = args
  else:
    (x_hbm, ei_hbm, s_out,
     eidx, rows, zbuf, acc, semz, semi, semg0, semg1, sems, semr) = args
    deg_out = ones_v = dacc = None
  semg = (semg0, semg1)
  c = lax.axis_index("c")
  s = lax.axis_index("s")
  wid = s * NC + c
  row0 = s * RPS
  base = wid * EPW

  ic = [pltpu.async_copy(
      ei_hbm.at[pl.ds(0, 2), pl.ds(base + j * CHUNK, CHUNK)], eidx.at[j], semi)
      for j in range(NCHUNK)]

  z16 = jnp.zeros((16,), jnp.float32)

  def zrow(i, _):
    for cc in range(U // 16):
      zbuf[i, pl.ds(cc * 16, 16)] = z16
    return 0

  lax.fori_loop(0, RPS, zrow, 0)
  if with_deg:
    o16 = jnp.ones((16,), jnp.float32)

    def orow(i, _):
      for cc in range(U // 16):
        ones_v[i, pl.ds(cc * 16, 16)] = o16
      return 0

    lax.fori_loop(0, CHUNK, orow, 0)

  zc = [pltpu.async_copy(zbuf, acc.at[pl.ds(row0, RPS)], semz)]
  if with_deg:
    zc.append(pltpu.async_copy(zbuf, dacc.at[pl.ds(row0, RPS)], semz))
  for cp in ic:
    cp.wait()
  gc = [pltpu.async_copy(x_hbm.at[eidx.at[j, 0]], rows.at[j], semg[j])
        for j in range(NCHUNK)]
  for cp in zc:
    cp.wait()
  plsc.subcore_barrier()
  sc = []
  for j in range(NCHUNK):
    gc[j].wait()
    sc.append(pltpu.async_copy(rows.at[j], acc.at[eidx.at[j, 1]], sems,
                               add=True))
    if with_deg:
      sc.append(pltpu.async_copy(ones_v, dacc.at[eidx.at[j, 1]], sems,
                                 add=True))
  for cp in sc:
    cp.wait()
  plsc.subcore_barrier()
  rc = [pltpu.async_copy(acc.at[pl.ds(row0, RPS)],
                         s_out.at[c, pl.ds(row0, RPS)], semr)]
  if with_deg:
    rc.append(pltpu.async_copy(dacc.at[pl.ds(row0, RPS)],
                               deg_out.at[c, pl.ds(row0, RPS)], semr))
  for cp in rc:
    cp.wait()


@functools.lru_cache(maxsize=None)
def _make_sc_agg(with_deg):
  out_type = [jax.ShapeDtypeStruct((NC, N, U), jnp.float32)]
  if with_deg:
    out_type.append(jax.ShapeDtypeStruct((NC, N, DW), jnp.float32))
  scratch = [
      pltpu.VMEM((NCHUNK, 2, CHUNK), jnp.int32),
      pltpu.VMEM((NCHUNK, CHUNK, U), jnp.float32),
  ]
  if with_deg:
    scratch.append(pltpu.VMEM((CHUNK, DW), jnp.float32))
  scratch.append(pltpu.VMEM((RPS, U), jnp.float32))
  scratch.append(pltpu.VMEM_SHARED((N, U), jnp.float32))
  if with_deg:
    scratch.append(pltpu.VMEM_SHARED((N, DW), jnp.float32))
  scratch.extend([pltpu.SemaphoreType.DMA] * (4 + NCHUNK))
  return pl.kernel(
      functools.partial(_sc_agg_body, with_deg),
      out_type=out_type,
      mesh=plsc.VectorSubcoreMesh(core_axis_name="c", subcore_axis_name="s",
                                  num_cores=NC, num_subcores=NS),
      scratch_types=scratch,
      name="sage_segment_sum" + ("_deg" if with_deg else ""),
  )


def _dotT(a, w, precision=lax.Precision.DEFAULT):
  return lax.dot_general(a, w, (((1,), (1,)), ((), ())),
                         preferred_element_type=jnp.float32,
                         precision=precision)


def _dotN(a, w, precision=lax.Precision.HIGHEST):
  return lax.dot_general(a, w, (((1,), (0,)), ((), ())),
                         preferred_element_type=jnp.float32,
                         precision=precision)


def _dense_first_body(s_ref, degw_ref, x_ref, wl_ref, bl_ref, wr_ref,
                      o_ref, deg_ref):
  deg = jnp.maximum(degw_ref[0, :, 0:1] + degw_ref[1, :, 0:1], 1.0)
  deg_ref[...] = deg
  mean = (s_ref[0] + s_ref[1]) / deg
  o_ref[...] = jnp.tanh(_dotT(mean, wl_ref[...]) + bl_ref[...]
                        + _dotT(x_ref[...], wr_ref[...]))


def _dense_body(s_ref, deg_ref, x_ref, wl_ref, bl_ref, wr_ref, o_ref):
  mean = (s_ref[0] + s_ref[1]) / deg_ref[...]
  o_ref[...] = jnp.tanh(_dotT(mean, wl_ref[...]) + bl_ref[...]
                        + _dotT(x_ref[...], wr_ref[...]))


def _final_body(s_ref, deg_ref, h_ref,
                wla_ref, bla_ref, wra_ref,
                wlc_ref, blc_ref, wrc_ref,
                wa_ref, wc_ref, bc_ref, rrep_ref, ttile_ref,
                ea_ref, ec_ref):
  mean = (s_ref[0] + s_ref[1]) / deg_ref[...]
  h = h_ref[...]
  xa = _dotT(mean, wla_ref[...]) + bla_ref[...] + _dotT(h, wra_ref[...])
  xc = _dotT(mean, wlc_ref[...]) + blc_ref[...] + _dotT(h, wrc_ref[...])
  wa = wa_ref[...]
  a = _dotT(xa, wa[:, :U])
  b4 = jnp.concatenate(
      [_dotT(wa[:, U:], xa[k * 128:(k + 1) * 128, :]) for k in range(4)],
      axis=0)
  cv = _dotT(xc, wc_ref[...])
  ma = jnp.max(a)
  mb = jnp.max(b4)
  lse = (ma + mb + jnp.log(jnp.sum(jnp.exp(a - ma)))
         + jnp.log(jnp.sum(jnp.exp(b4 - mb))))
  hi = lax.Precision.HIGHEST
  a_shift = a - lse
  a_rep = jnp.concatenate(
      [_dotN(rrep_ref[...], a_shift[128 * m:128 * (m + 1), :], hi)
       for m in range(4)], axis=0)
  b_tile = _dotN(ttile_ref[...], b4, hi)
  ea_ref[...] = a_rep + b_tile
  ec_ref[...] = jnp.tanh(jnp.mean(cv) + bc_ref[0, 0]).reshape(1, 1)


@functools.lru_cache(maxsize=None)
def _rrep():
  r = np.zeros((N, 128), np.float32)
  r[np.arange(N), np.arange(N) // 4] = 1.0
  return r


@functools.lru_cache(maxsize=None)
def _ttile():
  t = np.zeros((4 * N, 4), np.float32)
  t[np.arange(4 * N), np.arange(4 * N) % 4] = 1.0
  return t


def _dense(s_parts, deg, x, wl, bl, wr):
  return pl.pallas_call(
      _dense_body,
      out_shape=jax.ShapeDtypeStruct((N, U), jnp.float32),
  )(s_parts, deg, x, wl, bl.reshape(1, U), wr)


def kernel(x, edge_index, batch,
           Wl_first, bl_first, Wr_first,
           Wl_common, bl_common, Wr_common,
           Wl_actor, bl_actor, Wr_actor,
           Wl_critic, bl_critic, Wr_critic,
           Wa, ba, Wc, bc):
  ei = edge_index.astype(jnp.int32)

  s_x, degw = _make_sc_agg(True)(x, ei)
  h1, deg = pl.pallas_call(
      _dense_first_body,
      out_shape=[
          jax.ShapeDtypeStruct((N, U), jnp.float32),
          jax.ShapeDtypeStruct((N, 1), jnp.float32),
      ],
  )(s_x, degw, x, Wl_first, bl_first.reshape(1, U), Wr_first)
  (s_h1,) = _make_sc_agg(False)(h1, ei)
  h = _dense(s_h1, deg, h1, Wl_common, bl_common, Wr_common)
  (s_h,) = _make_sc_agg(False)(h, ei)

  ea, ec = pl.pallas_call(
      _final_body,
      out_shape=[
          jax.ShapeDtypeStruct((4 * N, 128), jnp.float32),
          jax.ShapeDtypeStruct((1, 1), jnp.float32),
      ],
  )(s_h, deg, h,
    Wl_actor, bl_actor.reshape(1, U), Wr_actor,
    Wl_critic, bl_critic.reshape(1, U), Wr_critic,
    Wa, Wc, bc.reshape(1, 1), _rrep(), _ttile())

  return ea.reshape(N * N, 1), ec


# speedup vs baseline: 8.9713x; 1.0029x over previous
import functools

import jax
import jax.numpy as jnp
import numpy as np
from jax import lax
from jax.experimental import pallas as pl
from jax.experimental.pallas import tpu as pltpu
from jax.experimental.pallas import tpu_sc as plsc

N = 512
U = 128
E = 8192

NC = 2
NS = 16
NW = NC * NS
EPW = E // NW
CHUNK = 128
NCHUNK = EPW // CHUNK
RPS = N // NS
DW = 128


DN = 16


def _sc_agg_body(with_deg, *args):
  if with_deg:
    (x_hbm, ei_hbm, s_out, deg_out,
     eidx, rows, ones_v, zbuf, acc, dacc,
     semz, semi, semg0, semg1, sems, semr) = args
  else:
    (x_hbm, ei_hbm, s_out,
     eidx, rows, zbuf, acc, semz, semi, semg0, semg1, sems, semr) = args
    deg_out = ones_v = dacc = None
  semg = (semg0, semg1)
  c = lax.axis_index("c")
  s = lax.axis_index("s")
  wid = s * NC + c
  row0 = s * RPS
  base = wid * EPW

  ic = [pltpu.async_copy(
      ei_hbm.at[pl.ds(0, 2), pl.ds(base + j * CHUNK, CHUNK)], eidx.at[j], semi)
      for j in range(NCHUNK)]

  z16 = jnp.zeros((16,), jnp.float32)

  def zrow(i, _):
    for cc in range(U // 16):
      zbuf[i, pl.ds(cc * 16, 16)] = z16
    return 0

  lax.fori_loop(0, RPS, zrow, 0)

  zc = [pltpu.async_copy(zbuf, acc.at[pl.ds(row0, RPS)], semz)]
  if with_deg:
    zc.append(pltpu.async_copy(zbuf, dacc.at[pl.ds(row0, RPS)], semz))
  for cp in ic:
    cp.wait()
  gc = [pltpu.async_copy(x_hbm.at[eidx.at[j, 0]], rows.at[j], semg[j])
        for j in range(NCHUNK)]
  if with_deg:
    o16 = jnp.ones((16,), jnp.float32)

    def orow(i, _):
      for cc in range(U // 16):
        ones_v[i, pl.ds(cc * 16, 16)] = o16
      return 0

    lax.fori_loop(0, CHUNK, orow, 0)
  for cp in zc:
    cp.wait()
  plsc.subcore_barrier()
  sc = []
  for j in range(NCHUNK):
    gc[j].wait()
    sc.append(pltpu.async_copy(rows.at[j], acc.at[eidx.at[j, 1]], sems,
                               add=True))
    if with_deg:
      sc.append(pltpu.async_copy(ones_v, dacc.at[eidx.at[j, 1]], sems,
                                 add=True))
  for cp in sc:
    cp.wait()
  plsc.subcore_barrier()
  rc = [pltpu.async_copy(acc.at[pl.ds(row0, RPS)],
                         s_out.at[c, pl.ds(row0, RPS)], semr)]
  if with_deg:
    rc.append(pltpu.async_copy(dacc.at[pl.ds(row0, RPS)],
                               deg_out.at[c, pl.ds(row0, RPS)], semr))
  for cp in rc:
    cp.wait()


@functools.lru_cache(maxsize=None)
def _make_sc_agg(with_deg):
  out_type = [jax.ShapeDtypeStruct((NC, N, U), jnp.float32)]
  if with_deg:
    out_type.append(jax.ShapeDtypeStruct((NC, N, DW), jnp.float32))
  scratch = [
      pltpu.VMEM((NCHUNK, 2, CHUNK), jnp.int32),
      pltpu.VMEM((NCHUNK, CHUNK, U), jnp.float32),
  ]
  if with_deg:
    scratch.append(pltpu.VMEM((CHUNK, DW), jnp.float32))
  scratch.append(pltpu.VMEM((RPS, U), jnp.float32))
  scratch.append(pltpu.VMEM_SHARED((N, U), jnp.float32))
  if with_deg:
    scratch.append(pltpu.VMEM_SHARED((N, DW), jnp.float32))
  scratch.extend([pltpu.SemaphoreType.DMA] * (4 + NCHUNK))
  return pl.kernel(
      functools.partial(_sc_agg_body, with_deg),
      out_type=out_type,
      mesh=plsc.VectorSubcoreMesh(core_axis_name="c", subcore_axis_name="s",
                                  num_cores=NC, num_subcores=NS),
      scratch_types=scratch,
      name="sage_segment_sum" + ("_deg" if with_deg else ""),
  )


def _dotT(a, w, precision=lax.Precision.DEFAULT):
  return lax.dot_general(a, w, (((1,), (1,)), ((), ())),
                         preferred_element_type=jnp.float32,
                         precision=precision)


def _dotN(a, w, precision=lax.Precision.HIGHEST):
  return lax.dot_general(a, w, (((1,), (0,)), ((), ())),
                         preferred_element_type=jnp.float32,
                         precision=precision)


def _dense_first_body(s_ref, degw_ref, x_ref, wl_ref, bl_ref, wr_ref,
                      o_ref, deg_ref):
  deg = jnp.maximum(degw_ref[0, :, 0:1] + degw_ref[1, :, 0:1], 1.0)
  deg_ref[...] = deg
  mean = (s_ref[0] + s_ref[1]) / deg
  o_ref[...] = jnp.tanh(_dotT(mean, wl_ref[...]) + bl_ref[...]
                        + _dotT(x_ref[...], wr_ref[...]))


def _dense_body(s_ref, deg_ref, x_ref, wl_ref, bl_ref, wr_ref, o_ref):
  mean = (s_ref[0] + s_ref[1]) / deg_ref[...]
  o_ref[...] = jnp.tanh(_dotT(mean, wl_ref[...]) + bl_ref[...]
                        + _dotT(x_ref[...], wr_ref[...]))


def _final_body(s_ref, deg_ref, h_ref,
                wla_ref, bla_ref, wra_ref,
                wlc_ref, blc_ref, wrc_ref,
                wa_ref, wc_ref, bc_ref, rrep_ref, ttile_ref,
                ea_ref, ec_ref):
  mean = (s_ref[0] + s_ref[1]) / deg_ref[...]
  h = h_ref[...]
  xa = _dotT(mean, wla_ref[...]) + bla_ref[...] + _dotT(h, wra_ref[...])
  xc = _dotT(mean, wlc_ref[...]) + blc_ref[...] + _dotT(h, wrc_ref[...])
  wa = wa_ref[...]
  a = _dotT(xa, wa[:, :U])
  b4 = jnp.concatenate(
      [_dotT(wa[:, U:], xa[k * 128:(k + 1) * 128, :]) for k in range(4)],
      axis=0)
  cv = _dotT(xc, wc_ref[...])
  ma = jnp.max(a)
  mb = jnp.max(b4)
  lse = (ma + mb + jnp.log(jnp.sum(jnp.exp(a - ma)))
         + jnp.log(jnp.sum(jnp.exp(b4 - mb))))
  hi = lax.Precision.HIGHEST
  a_shift = a - lse
  a_rep = jnp.concatenate(
      [_dotN(rrep_ref[...], a_shift[128 * m:128 * (m + 1), :], hi)
       for m in range(4)], axis=0)
  b_tile = _dotN(ttile_ref[...], b4, hi)
  ea_ref[...] = a_rep + b_tile
  ec_ref[...] = jnp.tanh(jnp.mean(cv) + bc_ref[0, 0]).reshape(1, 1)


@functools.lru_cache(maxsize=None)
def _rrep():
  r = np.zeros((N, 128), np.float32)
  r[np.arange(N), np.arange(N) // 4] = 1.0
  return r


@functools.lru_cache(maxsize=None)
def _ttile():
  t = np.zeros((4 * N, 4), np.float32)
  t[np.arange(4 * N), np.arange(4 * N) % 4] = 1.0
  return t


def _dense(s_parts, deg, x, wl, bl, wr):
  return pl.pallas_call(
      _dense_body,
      out_shape=jax.ShapeDtypeStruct((N, U), jnp.float32),
  )(s_parts, deg, x, wl, bl.reshape(1, U), wr)


def kernel(x, edge_index, batch,
           Wl_first, bl_first, Wr_first,
           Wl_common, bl_common, Wr_common,
           Wl_actor, bl_actor, Wr_actor,
           Wl_critic, bl_critic, Wr_critic,
           Wa, ba, Wc, bc):
  ei = edge_index.astype(jnp.int32)

  s_x, degw = _make_sc_agg(True)(x, ei)
  h1, deg = pl.pallas_call(
      _dense_first_body,
      out_shape=[
          jax.ShapeDtypeStruct((N, U), jnp.float32),
          jax.ShapeDtypeStruct((N, 1), jnp.float32),
      ],
  )(s_x, degw, x, Wl_first, bl_first.reshape(1, U), Wr_first)
  (s_h1,) = _make_sc_agg(False)(h1, ei)
  h = _dense(s_h1, deg, h1, Wl_common, bl_common, Wr_common)
  (s_h,) = _make_sc_agg(False)(h, ei)

  ea, ec = pl.pallas_call(
      _final_body,
      out_shape=[
          jax.ShapeDtypeStruct((4 * N, 128), jnp.float32),
          jax.ShapeDtypeStruct((1, 1), jnp.float32),
      ],
  )(s_h, deg, h,
    Wl_actor, bl_actor.reshape(1, U), Wr_actor,
    Wl_critic, bl_critic.reshape(1, U), Wr_critic,
    Wa, Wc, bc.reshape(1, 1), _rrep(), _ttile())

  return ea.reshape(N * N, 1), ec


# speedup vs baseline: 8.9795x; 1.0009x over previous
import functools

import jax
import jax.numpy as jnp
import numpy as np
from jax import lax
from jax.experimental import pallas as pl
from jax.experimental.pallas import tpu as pltpu
from jax.experimental.pallas import tpu_sc as plsc

N = 512
U = 128
E = 8192

NC = 2
NS = 16
NW = NC * NS
EPW = E // NW
CHUNK = 128
NCHUNK = EPW // CHUNK
RPS = N // NS
DW = 128


def _sc_agg_body(with_deg, *args):
  if with_deg:
    (x_hbm, ei_hbm, s_out, deg_out,
     eidx, rows, ones_v, zbuf, acc, dacc,
     semz, semi, semg0, semg1, sems, semr) = args
  else:
    (x_hbm, ei_hbm, s_out,
     eidx, rows, zbuf, acc, semz, semi, semg0, semg1, sems, semr) = args
    deg_out = ones_v = dacc = None
  semg = (semg0, semg1)
  c = lax.axis_index("c")
  s = lax.axis_index("s")
  wid = s * NC + c
  row0 = s * RPS
  base = wid * EPW

  ic = [pltpu.async_copy(
      ei_hbm.at[pl.ds(0, 2), pl.ds(base + j * CHUNK, CHUNK)], eidx.at[j], semi)
      for j in range(NCHUNK)]

  z16 = jnp.zeros((16,), jnp.float32)

  def zrow(i, _):
    for cc in range(U // 16):
      zbuf[i, pl.ds(cc * 16, 16)] = z16
    return 0

  lax.fori_loop(0, RPS, zrow, 0)

  zc = [pltpu.async_copy(zbuf, acc.at[pl.ds(row0, RPS)], semz)]
  if with_deg:
    zc.append(pltpu.async_copy(zbuf, dacc.at[pl.ds(row0, RPS)], semz))
  for cp in ic:
    cp.wait()
  gc = [pltpu.async_copy(x_hbm.at[eidx.at[j, 0]], rows.at[j], semg[j])
        for j in range(NCHUNK)]
  if with_deg:
    o16 = jnp.ones((16,), jnp.float32)

    def orow(i, _):
      for cc in range(U // 16):
        ones_v[i, pl.ds(cc * 16, 16)] = o16
      return 0

    lax.fori_loop(0, CHUNK, orow, 0)
  for cp in zc:
    cp.wait()
  plsc.subcore_barrier()
  sc = []
  for j in range(NCHUNK):
    gc[j].wait()
    sc.append(pltpu.async_copy(rows.at[j], acc.at[eidx.at[j, 1]], sems,
                               add=True))
    if with_deg:
      sc.append(pltpu.async_copy(ones_v, dacc.at[eidx.at[j, 1]], sems,
                                 add=True))
  for cp in sc:
    cp.wait()
  plsc.subcore_barrier()
  rc = [pltpu.async_copy(acc.at[pl.ds(row0, RPS)],
                         s_out.at[c, pl.ds(row0, RPS)], semr)]
  if with_deg:
    rc.append(pltpu.async_copy(dacc.at[pl.ds(row0, RPS)],
                               deg_out.at[c, pl.ds(row0, RPS)], semr))
  for cp in rc:
    cp.wait()


@functools.lru_cache(maxsize=None)
def _make_sc_agg(with_deg):
  out_type = [jax.ShapeDtypeStruct((NC, N, U), jnp.float32)]
  if with_deg:
    out_type.append(jax.ShapeDtypeStruct((NC, N, DW), jnp.float32))
  scratch = [
      pltpu.VMEM((NCHUNK, 2, CHUNK), jnp.int32),
      pltpu.VMEM((NCHUNK, CHUNK, U), jnp.float32),
  ]
  if with_deg:
    scratch.append(pltpu.VMEM((CHUNK, DW), jnp.float32))
  scratch.append(pltpu.VMEM((RPS, U), jnp.float32))
  scratch.append(pltpu.VMEM_SHARED((N, U), jnp.float32))
  if with_deg:
    scratch.append(pltpu.VMEM_SHARED((N, DW), jnp.float32))
  scratch.extend([pltpu.SemaphoreType.DMA] * (4 + NCHUNK))
  return pl.kernel(
      functools.partial(_sc_agg_body, with_deg),
      out_type=out_type,
      mesh=plsc.VectorSubcoreMesh(core_axis_name="c", subcore_axis_name="s",
                                  num_cores=NC, num_subcores=NS),
      scratch_types=scratch,
      name="sage_segment_sum" + ("_deg" if with_deg else ""),
  )


def _dotT(a, w, precision=lax.Precision.DEFAULT):
  return lax.dot_general(a, w, (((1,), (1,)), ((), ())),
                         preferred_element_type=jnp.float32,
                         precision=precision)


def _dotN(a, w, precision=lax.Precision.HIGHEST):
  return lax.dot_general(a, w, (((1,), (0,)), ((), ())),
                         preferred_element_type=jnp.float32,
                         precision=precision)


def _dense_first_body(s_ref, degw_ref, x_ref, wl_ref, bl_ref, wr_ref,
                      o_ref, deg_ref):
  deg = jnp.maximum(degw_ref[0, :, 0:1] + degw_ref[1, :, 0:1], 1.0)
  deg_ref[...] = deg
  mean = (s_ref[0] + s_ref[1]) / deg
  o_ref[...] = jnp.tanh(_dotT(mean, wl_ref[...]) + bl_ref[...]
                        + _dotT(x_ref[...], wr_ref[...]))


def _dense_body(s_ref, deg_ref, x_ref, wl_ref, bl_ref, wr_ref, o_ref):
  mean = (s_ref[0] + s_ref[1]) / deg_ref[...]
  o_ref[...] = jnp.tanh(_dotT(mean, wl_ref[...]) + bl_ref[...]
                        + _dotT(x_ref[...], wr_ref[...]))


def _final_body(s_ref, deg_ref, h_ref,
                wla_ref, bla_ref, wra_ref,
                wlc_ref, blc_ref, wrc_ref,
                wa_ref, wc_ref, bc_ref, rrep_ref, ttile_ref,
                ea_ref, ec_ref):
  mean = (s_ref[0] + s_ref[1]) / deg_ref[...]
  h = h_ref[...]
  xa = _dotT(mean, wla_ref[...]) + bla_ref[...] + _dotT(h, wra_ref[...])
  xc = _dotT(mean, wlc_ref[...]) + blc_ref[...] + _dotT(h, wrc_ref[...])
  wa = wa_ref[...]
  a = _dotT(xa, wa[:, :U])
  b4 = jnp.concatenate(
      [_dotT(wa[:, U:], xa[k * 128:(k + 1) * 128, :]) for k in range(4)],
      axis=0)
  cv = _dotT(xc, wc_ref[...])
  ma = jnp.max(a)
  mb = jnp.max(b4)
  lse = (ma + mb + jnp.log(jnp.sum(jnp.exp(a - ma)))
         + jnp.log(jnp.sum(jnp.exp(b4 - mb))))
  hi = lax.Precision.HIGHEST
  a_shift = a - lse
  a_rep = jnp.concatenate(
      [_dotN(rrep_ref[...], a_shift[128 * m:128 * (m + 1), :], hi)
       for m in range(4)], axis=0)
  b_tile = _dotN(ttile_ref[...], b4, hi)
  ea_ref[...] = a_rep + b_tile
  ec_ref[...] = jnp.tanh(jnp.mean(cv) + bc_ref[0, 0]).reshape(1, 1)


@functools.lru_cache(maxsize=None)
def _rrep():
  r = np.zeros((N, 128), np.float32)
  r[np.arange(N), np.arange(N) // 4] = 1.0
  return r


@functools.lru_cache(maxsize=None)
def _ttile():
  t = np.zeros((4 * N, 4), np.float32)
  t[np.arange(4 * N), np.arange(4 * N) % 4] = 1.0
  return t


def _dense(s_parts, deg, x, wl, bl, wr):
  return pl.pallas_call(
      _dense_body,
      out_shape=jax.ShapeDtypeStruct((N, U), jnp.float32),
  )(s_parts, deg, x, wl, bl.reshape(1, U), wr)


def kernel(x, edge_index, batch,
           Wl_first, bl_first, Wr_first,
           Wl_common, bl_common, Wr_common,
           Wl_actor, bl_actor, Wr_actor,
           Wl_critic, bl_critic, Wr_critic,
           Wa, ba, Wc, bc):
  ei = edge_index.astype(jnp.int32)

  s_x, degw = _make_sc_agg(True)(x, ei)
  h1, deg = pl.pallas_call(
      _dense_first_body,
      out_shape=[
          jax.ShapeDtypeStruct((N, U), jnp.float32),
          jax.ShapeDtypeStruct((N, 1), jnp.float32),
      ],
  )(s_x, degw, x, Wl_first, bl_first.reshape(1, U), Wr_first)
  (s_h1,) = _make_sc_agg(False)(h1, ei)
  h = _dense(s_h1, deg, h1, Wl_common, bl_common, Wr_common)
  (s_h,) = _make_sc_agg(False)(h, ei)

  ea, ec = pl.pallas_call(
      _final_body,
      out_shape=[
          jax.ShapeDtypeStruct((4 * N, 128), jnp.float32),
          jax.ShapeDtypeStruct((1, 1), jnp.float32),
      ],
  )(s_h, deg, h,
    Wl_actor, bl_actor.reshape(1, U), Wr_actor,
    Wl_critic, bl_critic.reshape(1, U), Wr_critic,
    Wa, Wc, bc.reshape(1, 1), _rrep(), _ttile())

  return ea.reshape(N * N, 1), ec
